# baseline (speedup 1.0000x reference)
import functools

import jax
import jax.numpy as jnp
from jax import lax
from jax.experimental import pallas as pl
from jax.experimental.pallas import tpu as pltpu
from jax.experimental.pallas import tpu_sc as plsc

NN = 10000
EE = 320000
DD = 128
NEXP = 8
NC = 2
NS = 16
NW = NC * NS
EPW = EE // NW
CHUNK = 96
EPWP = 10080
NCHUNK = EPWP // CHUNK
GRP = 21
NGRP = NCHUNK // GRP
NNP = 10240
RPT = NNP // NS
DEG_W = 16
RB = 1000


def _mesh():
    return plsc.VectorSubcoreMesh(core_axis_name="c", subcore_axis_name="s")


def _sc_degree(dst_idx, zeros16):
    @functools.partial(
        pl.kernel,
        out_type=jax.ShapeDtypeStruct((NC, NNP, DEG_W), jnp.float32),
        mesh=_mesh(),
        scratch_types=[
            pltpu.VMEM((GRP, CHUNK), jnp.int32),
            pltpu.VMEM((CHUNK, DEG_W), jnp.float32),
            pltpu.VMEM_SHARED((NNP, DEG_W), jnp.float32),
            pltpu.SemaphoreType.DMA,
        ],
    )
    def k(dst_hbm, zeros_hbm, out_hbm, idx_v, ones_v, acc_sh, dsem):
        cid = lax.axis_index("c")
        sid = lax.axis_index("s")
        wid = sid * NC + cid
        one16 = jnp.full((DEG_W,), 1.0, dtype=jnp.float32)
        for i in range(CHUNK):
            ones_v[i, :] = one16
        pltpu.sync_copy(zeros_hbm.at[pl.ds(sid * RPT, RPT)],
                        acc_sh.at[pl.ds(sid * RPT, RPT)])
        plsc.subcore_barrier()

        def group(g, carry):
            pltpu.sync_copy(dst_hbm.at[wid, g], idx_v)

            def body(c, carry2):
                pltpu.sync_copy(ones_v, acc_sh.at[idx_v.at[c]], add=True)
                return carry2

            lax.fori_loop(0, GRP, body, 0)
            return carry

        lax.fori_loop(0, NGRP, group, 0)
        plsc.subcore_barrier()
        pltpu.sync_copy(acc_sh.at[pl.ds(sid * RPT, RPT)],
                        out_hbm.at[cid, pl.ds(sid * RPT, RPT)])

    return k(dst_idx, zeros16)


def _sc_scatter(src_idx, dst_idx, table, zeros128):
    @functools.partial(
        pl.kernel,
        out_type=jax.ShapeDtypeStruct((NC, NNP, DD), jnp.float32),
        mesh=_mesh(),
        scratch_types=[
            pltpu.VMEM((GRP, CHUNK), jnp.int32),
            pltpu.VMEM((GRP, CHUNK), jnp.int32),
            pltpu.VMEM((2, CHUNK, DD), jnp.float32),
            pltpu.VMEM_SHARED((NNP, DD), jnp.float32),
            pltpu.SemaphoreType.DMA((2,)),
        ],
    )
    def k(src_hbm, dst_hbm, tab_hbm, zeros_hbm, out_hbm,
          srcv, dstv, rows, acc_sh, gsem):
        cid = lax.axis_index("c")
        sid = lax.axis_index("s")
        wid = sid * NC + cid
        pltpu.sync_copy(zeros_hbm.at[pl.ds(sid * RPT, RPT)],
                        acc_sh.at[pl.ds(sid * RPT, RPT)])
        plsc.subcore_barrier()

        def group(g, carry):
            pltpu.sync_copy(src_hbm.at[wid, g], srcv)
            pltpu.sync_copy(dst_hbm.at[wid, g], dstv)
            pltpu.async_copy(tab_hbm.at[srcv.at[0]], rows.at[0], gsem.at[0])

            def body(c, carry2):
                par = lax.rem(c, 2)
                nxt = lax.rem(c + 1, 2)

                @pl.when(c < GRP - 1)
                def _():
                    pltpu.async_copy(tab_hbm.at[srcv.at[c + 1]], rows.at[nxt],
                                     gsem.at[nxt])

                pltpu.make_async_copy(tab_hbm.at[srcv.at[c]], rows.at[par],
                                      gsem.at[par]).wait()
                pltpu.sync_copy(rows.at[par], acc_sh.at[dstv.at[c]], add=True)
                return carry2

            lax.fori_loop(0, GRP, body, 0)
            return carry

        lax.fori_loop(0, NGRP, group, 0)
        plsc.subcore_barrier()
        pltpu.sync_copy(acc_sh.at[pl.ds(sid * RPT, RPT)],
                        out_hbm.at[cid, pl.ds(sid * RPT, RPT)])

    return k(src_idx, dst_idx, table, zeros128)


def _moe_block(x, wg, W, b):
    r = x.shape[0]
    logits = jnp.dot(x, wg, preferred_element_type=jnp.float32)
    iota = lax.broadcasted_iota(jnp.int32, (r, NEXP), 1)
    v1 = jnp.max(logits, axis=1, keepdims=True)
    i1 = jnp.min(jnp.where(logits == v1, iota, NEXP), axis=1, keepdims=True)
    l2 = jnp.where(iota == i1, -jnp.inf, logits)
    v2 = jnp.max(l2, axis=1, keepdims=True)
    i2 = jnp.min(jnp.where(l2 == v2, iota, NEXP), axis=1, keepdims=True)
    a = 1.0 / (1.0 + jnp.exp(v2 - v1))
    gates = (jnp.where(iota == i1, a, 0.0)
             + jnp.where(iota == i2, 1.0 - a, 0.0))
    acc = jnp.zeros((r, DD), jnp.float32)
    for e in range(NEXP):
        y = jnp.dot(x, W[e], preferred_element_type=jnp.float32) + b[e][None, :]
        acc = acc + gates[:, e:e + 1] * y
    return acc


def _tc_moe1(x, wg, W, b, d0, d1):
    def body(x_ref, wg_ref, W_ref, b_ref, d0_ref, d1_ref, mp_ref, dinv_ref):
        deg = d0_ref[...] + d1_ref[...] + 1.0
        dinv = lax.rsqrt(deg)
        m = _moe_block(x_ref[...], wg_ref[...], W_ref[...], b_ref[...])
        mp_ref[...] = m * dinv
        dinv_ref[...] = dinv

    grid = (NN // RB,)
    return pl.pallas_call(
        body,
        grid=grid,
        in_specs=[
            pl.BlockSpec((RB, DD), lambda i: (i, 0)),
            pl.BlockSpec((DD, NEXP), lambda i: (0, 0)),
            pl.BlockSpec((NEXP, DD, DD), lambda i: (0, 0, 0)),
            pl.BlockSpec((NEXP, DD), lambda i: (0, 0)),
            pl.BlockSpec((RB, 1), lambda i: (i, 0)),
            pl.BlockSpec((RB, 1), lambda i: (i, 0)),
        ],
        out_specs=[
            pl.BlockSpec((RB, DD), lambda i: (i, 0)),
            pl.BlockSpec((RB, 1), lambda i: (i, 0)),
        ],
        out_shape=[
            jax.ShapeDtypeStruct((NN, DD), jnp.float32),
            jax.ShapeDtypeStruct((NN, 1), jnp.float32),
        ],
    )(x, wg, W, b, d0, d1)


def _tc_moe2(p0, p1, mp1, dinv, wg, W, b):
    def body(p0_ref, p1_ref, mp1_ref, dinv_ref, wg_ref, W_ref, b_ref,
             out1_ref, mp2_ref):
        dinv = dinv_ref[...]
        out1 = (p0_ref[...] + p1_ref[...] + mp1_ref[...]) * dinv
        out1_ref[...] = out1
        m2 = _moe_block(out1, wg_ref[...], W_ref[...], b_ref[...])
        mp2_ref[...] = m2 * dinv

    grid = (NN // RB,)
    return pl.pallas_call(
        body,
        grid=grid,
        in_specs=[
            pl.BlockSpec((RB, DD), lambda i: (i, 0)),
            pl.BlockSpec((RB, DD), lambda i: (i, 0)),
            pl.BlockSpec((RB, DD), lambda i: (i, 0)),
            pl.BlockSpec((RB, 1), lambda i: (i, 0)),
            pl.BlockSpec((DD, NEXP), lambda i: (0, 0)),
            pl.BlockSpec((NEXP, DD, DD), lambda i: (0, 0, 0)),
            pl.BlockSpec((NEXP, DD), lambda i: (0, 0)),
        ],
        out_specs=[
            pl.BlockSpec((RB, DD), lambda i: (i, 0)),
            pl.BlockSpec((RB, DD), lambda i: (i, 0)),
        ],
        out_shape=[
            jax.ShapeDtypeStruct((NN, DD), jnp.float32),
            jax.ShapeDtypeStruct((NN, DD), jnp.float32),
        ],
    )(p0, p1, mp1, dinv, wg, W, b)


def _tc_comb(p0, p1, mp2, dinv):
    def body(p0_ref, p1_ref, mp2_ref, dinv_ref, out_ref):
        out_ref[...] = (p0_ref[...] + p1_ref[...] + mp2_ref[...]) * dinv_ref[...]

    grid = (NN // RB,)
    return pl.pallas_call(
        body,
        grid=grid,
        in_specs=[
            pl.BlockSpec((RB, DD), lambda i: (i, 0)),
            pl.BlockSpec((RB, DD), lambda i: (i, 0)),
            pl.BlockSpec((RB, DD), lambda i: (i, 0)),
            pl.BlockSpec((RB, 1), lambda i: (i, 0)),
        ],
        out_specs=pl.BlockSpec((RB, DD), lambda i: (i, 0)),
        out_shape=jax.ShapeDtypeStruct((NN, DD), jnp.float32),
    )(p0, p1, mp2, dinv)


def kernel(x, pos_edge_index, w_gate1, W1, b1, w_gate2, W2, b2):
    pad = EPWP - EPW
    src = jnp.pad(pos_edge_index[0].reshape(NW, EPW), ((0, 0), (0, pad)),
                  constant_values=0).reshape(NW, NGRP, GRP, CHUNK)
    pad_dst = jnp.broadcast_to(NN + (jnp.arange(pad, dtype=jnp.int32)
                                     % (NNP - NN)), (NW, pad))
    dst = jnp.concatenate(
        [pos_edge_index[1].reshape(NW, EPW), pad_dst], axis=1,
    ).reshape(NW, NGRP, GRP, CHUNK)
    zeros128 = jnp.zeros((NNP, DD), jnp.float32)
    zeros16 = jnp.zeros((NNP, DEG_W), jnp.float32)

    degp = _sc_degree(dst, zeros16)
    d0 = degp[0, :NN, :1]
    d1 = degp[1, :NN, :1]

    mp1, dinv = _tc_moe1(x, w_gate1, W1, b1, d0, d1)
    p1 = _sc_scatter(src, dst, mp1, zeros128)
    out1, mp2 = _tc_moe2(p1[0, :NN], p1[1, :NN], mp1, dinv, w_gate2, W2, b2)
    p2 = _sc_scatter(src, dst, mp2, zeros128)
    out2 = _tc_comb(p2[0, :NN], p2[1, :NN], mp2, dinv)
    return (out2, out1, out2)


# speedup vs baseline: 1.4428x; 1.4428x over previous
import functools

import jax
import jax.numpy as jnp
from jax import lax
from jax.experimental import pallas as pl
from jax.experimental.pallas import tpu as pltpu
from jax.experimental.pallas import tpu_sc as plsc

NN = 10000
EE = 320000
DD = 128
NEXP = 8
NC = 2
NS = 16
NW = NC * NS
EPW = EE // NW
CHUNK = 80
NCHUNK = EPW // CHUNK
GRP = 25
NGRP = NCHUNK // GRP
NNP = 10240
RPT = NNP // NS
DEG_W = 16
RB = 1000


def _mesh():
    return plsc.VectorSubcoreMesh(core_axis_name="c", subcore_axis_name="s")


def _sc_degree(dst_idx, zeros16):
    @functools.partial(
        pl.kernel,
        out_type=jax.ShapeDtypeStruct((NC, NNP, DEG_W), jnp.float32),
        mesh=_mesh(),
        scratch_types=[
            pltpu.VMEM((GRP, CHUNK), jnp.int32),
            pltpu.VMEM((CHUNK, DEG_W), jnp.float32),
            pltpu.VMEM_SHARED((NNP, DEG_W), jnp.float32),
            pltpu.SemaphoreType.DMA,
        ],
    )
    def k(dst_hbm, zeros_hbm, out_hbm, idx_v, ones_v, acc_sh, dsem):
        cid = lax.axis_index("c")
        sid = lax.axis_index("s")
        wid = sid * NC + cid
        one16 = jnp.full((DEG_W,), 1.0, dtype=jnp.float32)
        for i in range(CHUNK):
            ones_v[i, :] = one16
        pltpu.sync_copy(zeros_hbm.at[pl.ds(sid * RPT, RPT)],
                        acc_sh.at[pl.ds(sid * RPT, RPT)])
        plsc.subcore_barrier()

        def group(g, carry):
            pltpu.sync_copy(dst_hbm.at[wid, g], idx_v)

            def fire(c, carry2):
                pltpu.async_copy(ones_v, acc_sh.at[idx_v.at[c]], dsem,
                                 add=True)
                return carry2

            lax.fori_loop(0, GRP, fire, 0)

            def drain(c, carry2):
                pltpu.make_async_copy(ones_v, acc_sh.at[idx_v.at[c]],
                                      dsem).wait()
                return carry2

            lax.fori_loop(0, GRP, drain, 0)
            return carry

        lax.fori_loop(0, NGRP, group, 0)
        plsc.subcore_barrier()
        pltpu.sync_copy(acc_sh.at[pl.ds(sid * RPT, RPT)],
                        out_hbm.at[cid, pl.ds(sid * RPT, RPT)])

    return k(dst_idx, zeros16)


def _sc_scatter(src_idx, dst_idx, table, zeros128):
    @functools.partial(
        pl.kernel,
        out_type=jax.ShapeDtypeStruct((NC, NNP, DD), jnp.float32),
        mesh=_mesh(),
        scratch_types=[
            pltpu.VMEM((GRP, CHUNK), jnp.int32),
            pltpu.VMEM((GRP, CHUNK), jnp.int32),
            pltpu.VMEM((2, CHUNK, DD), jnp.float32),
            pltpu.VMEM_SHARED((NNP, DD), jnp.float32),
            pltpu.SemaphoreType.DMA((2,)),
        ],
    )
    def k(src_hbm, dst_hbm, tab_hbm, zeros_hbm, out_hbm,
          srcv, dstv, rows, acc_sh, gsem):
        cid = lax.axis_index("c")
        sid = lax.axis_index("s")
        wid = sid * NC + cid
        pltpu.sync_copy(zeros_hbm.at[pl.ds(sid * RPT, RPT)],
                        acc_sh.at[pl.ds(sid * RPT, RPT)])
        plsc.subcore_barrier()

        def group(g, carry):
            pltpu.sync_copy(src_hbm.at[wid, g], srcv)
            pltpu.sync_copy(dst_hbm.at[wid, g], dstv)
            pltpu.async_copy(tab_hbm.at[srcv.at[0]], rows.at[0], gsem.at[0])

            def body(c, carry2):
                par = lax.rem(c, 2)
                nxt = lax.rem(c + 1, 2)

                @pl.when(c < GRP - 1)
                def _():
                    pltpu.async_copy(tab_hbm.at[srcv.at[c + 1]], rows.at[nxt],
                                     gsem.at[nxt])

                pltpu.make_async_copy(tab_hbm.at[srcv.at[c]], rows.at[par],
                                      gsem.at[par]).wait()
                pltpu.sync_copy(rows.at[par], acc_sh.at[dstv.at[c]], add=True)
                return carry2

            lax.fori_loop(0, GRP, body, 0)
            return carry

        lax.fori_loop(0, NGRP, group, 0)
        plsc.subcore_barrier()
        pltpu.sync_copy(acc_sh.at[pl.ds(sid * RPT, RPT)],
                        out_hbm.at[cid, pl.ds(sid * RPT, RPT)])

    return k(src_idx, dst_idx, table, zeros128)


def _moe_block(x, wg, W, b):
    r = x.shape[0]
    logits = jnp.dot(x, wg, preferred_element_type=jnp.float32)
    iota = lax.broadcasted_iota(jnp.int32, (r, NEXP), 1)
    v1 = jnp.max(logits, axis=1, keepdims=True)
    i1 = jnp.min(jnp.where(logits == v1, iota, NEXP), axis=1, keepdims=True)
    l2 = jnp.where(iota == i1, -jnp.inf, logits)
    v2 = jnp.max(l2, axis=1, keepdims=True)
    i2 = jnp.min(jnp.where(l2 == v2, iota, NEXP), axis=1, keepdims=True)
    a = 1.0 / (1.0 + jnp.exp(v2 - v1))
    gates = (jnp.where(iota == i1, a, 0.0)
             + jnp.where(iota == i2, 1.0 - a, 0.0))
    acc = jnp.zeros((r, DD), jnp.float32)
    for e in range(NEXP):
        y = jnp.dot(x, W[e], preferred_element_type=jnp.float32) + b[e][None, :]
        acc = acc + gates[:, e:e + 1] * y
    return acc


def _tc_moe1(x, wg, W, b, d0, d1):
    def body(x_ref, wg_ref, W_ref, b_ref, d0_ref, d1_ref, mp_ref, dinv_ref):
        deg = d0_ref[...] + d1_ref[...] + 1.0
        dinv = lax.rsqrt(deg)
        m = _moe_block(x_ref[...], wg_ref[...], W_ref[...], b_ref[...])
        mp_ref[...] = m * dinv
        dinv_ref[...] = dinv

    grid = (NN // RB,)
    return pl.pallas_call(
        body,
        grid=grid,
        in_specs=[
            pl.BlockSpec((RB, DD), lambda i: (i, 0)),
            pl.BlockSpec((DD, NEXP), lambda i: (0, 0)),
            pl.BlockSpec((NEXP, DD, DD), lambda i: (0, 0, 0)),
            pl.BlockSpec((NEXP, DD), lambda i: (0, 0)),
            pl.BlockSpec((RB, 1), lambda i: (i, 0)),
            pl.BlockSpec((RB, 1), lambda i: (i, 0)),
        ],
        out_specs=[
            pl.BlockSpec((RB, DD), lambda i: (i, 0)),
            pl.BlockSpec((RB, 1), lambda i: (i, 0)),
        ],
        out_shape=[
            jax.ShapeDtypeStruct((NN, DD), jnp.float32),
            jax.ShapeDtypeStruct((NN, 1), jnp.float32),
        ],
    )(x, wg, W, b, d0, d1)


def _tc_moe2(p0, p1, mp1, dinv, wg, W, b):
    def body(p0_ref, p1_ref, mp1_ref, dinv_ref, wg_ref, W_ref, b_ref,
             out1_ref, mp2_ref):
        dinv = dinv_ref[...]
        out1 = (p0_ref[...] + p1_ref[...] + mp1_ref[...]) * dinv
        out1_ref[...] = out1
        m2 = _moe_block(out1, wg_ref[...], W_ref[...], b_ref[...])
        mp2_ref[...] = m2 * dinv

    grid = (NN // RB,)
    return pl.pallas_call(
        body,
        grid=grid,
        in_specs=[
            pl.BlockSpec((RB, DD), lambda i: (i, 0)),
            pl.BlockSpec((RB, DD), lambda i: (i, 0)),
            pl.BlockSpec((RB, DD), lambda i: (i, 0)),
            pl.BlockSpec((RB, 1), lambda i: (i, 0)),
            pl.BlockSpec((DD, NEXP), lambda i: (0, 0)),
            pl.BlockSpec((NEXP, DD, DD), lambda i: (0, 0, 0)),
            pl.BlockSpec((NEXP, DD), lambda i: (0, 0)),
        ],
        out_specs=[
            pl.BlockSpec((RB, DD), lambda i: (i, 0)),
            pl.BlockSpec((RB, DD), lambda i: (i, 0)),
        ],
        out_shape=[
            jax.ShapeDtypeStruct((NN, DD), jnp.float32),
            jax.ShapeDtypeStruct((NN, DD), jnp.float32),
        ],
    )(p0, p1, mp1, dinv, wg, W, b)


def _tc_comb(p0, p1, mp2, dinv):
    def body(p0_ref, p1_ref, mp2_ref, dinv_ref, out_ref):
        out_ref[...] = (p0_ref[...] + p1_ref[...] + mp2_ref[...]) * dinv_ref[...]

    grid = (NN // RB,)
    return pl.pallas_call(
        body,
        grid=grid,
        in_specs=[
            pl.BlockSpec((RB, DD), lambda i: (i, 0)),
            pl.BlockSpec((RB, DD), lambda i: (i, 0)),
            pl.BlockSpec((RB, DD), lambda i: (i, 0)),
            pl.BlockSpec((RB, 1), lambda i: (i, 0)),
        ],
        out_specs=pl.BlockSpec((RB, DD), lambda i: (i, 0)),
        out_shape=jax.ShapeDtypeStruct((NN, DD), jnp.float32),
    )(p0, p1, mp2, dinv)


def kernel(x, pos_edge_index, w_gate1, W1, b1, w_gate2, W2, b2):
    src = pos_edge_index[0].reshape(NW, NGRP, GRP, CHUNK)
    dst = pos_edge_index[1].reshape(NW, NGRP, GRP, CHUNK)
    zeros128 = jnp.zeros((NNP, DD), jnp.float32)
    zeros16 = jnp.zeros((NNP, DEG_W), jnp.float32)

    degp = _sc_degree(dst, zeros16)
    d0 = degp[0, :NN, :1]
    d1 = degp[1, :NN, :1]

    mp1, dinv = _tc_moe1(x, w_gate1, W1, b1, d0, d1)
    p1 = _sc_scatter(src, dst, mp1, zeros128)
    out1, mp2 = _tc_moe2(p1[0, :NN], p1[1, :NN], mp1, dinv, w_gate2, W2, b2)
    p2 = _sc_scatter(src, dst, mp2, zeros128)
    out2 = _tc_comb(p2[0, :NN], p2[1, :NN], mp2, dinv)
    return (out2, out1, out2)


# speedup vs baseline: 1.4876x; 1.0310x over previous
import functools

import jax
import jax.numpy as jnp
from jax import lax
from jax.experimental import pallas as pl
from jax.experimental.pallas import tpu as pltpu
from jax.experimental.pallas import tpu_sc as plsc

NN = 10000
EE = 320000
DD = 128
NEXP = 8
NC = 2
NS = 16
NW = NC * NS
EPW = EE // NW
CHUNK = 80
NCHUNK = EPW // CHUNK
GRP = 25
NGRP = NCHUNK // GRP
NNP = 10240
RPT = NNP // NS
DEG_W = 16
RB = 1000


def _mesh():
    return plsc.VectorSubcoreMesh(core_axis_name="c", subcore_axis_name="s")


def _sc_degree(dst_idx, zeros16):
    @functools.partial(
        pl.kernel,
        out_type=jax.ShapeDtypeStruct((NC, NNP, DEG_W), jnp.float32),
        mesh=_mesh(),
        scratch_types=[
            pltpu.VMEM((GRP, CHUNK), jnp.int32),
            pltpu.VMEM((CHUNK, DEG_W), jnp.float32),
            pltpu.VMEM_SHARED((NNP, DEG_W), jnp.float32),
            pltpu.SemaphoreType.DMA,
        ],
    )
    def k(dst_hbm, zeros_hbm, out_hbm, idx_v, ones_v, acc_sh, dsem):
        cid = lax.axis_index("c")
        sid = lax.axis_index("s")
        wid = sid * NC + cid
        one16 = jnp.full((DEG_W,), 1.0, dtype=jnp.float32)
        for i in range(CHUNK):
            ones_v[i, :] = one16
        pltpu.sync_copy(zeros_hbm.at[pl.ds(sid * RPT, RPT)],
                        acc_sh.at[pl.ds(sid * RPT, RPT)])
        plsc.subcore_barrier()

        def group(g, carry):
            pltpu.sync_copy(dst_hbm.at[wid, g], idx_v)

            def fire(c, carry2):
                pltpu.async_copy(ones_v, acc_sh.at[idx_v.at[c]], dsem,
                                 add=True)
                return carry2

            lax.fori_loop(0, GRP, fire, 0)

            def drain(c, carry2):
                pltpu.make_async_copy(ones_v, acc_sh.at[idx_v.at[c]],
                                      dsem).wait()
                return carry2

            lax.fori_loop(0, GRP, drain, 0)
            return carry

        lax.fori_loop(0, NGRP, group, 0)
        plsc.subcore_barrier()
        pltpu.sync_copy(acc_sh.at[pl.ds(sid * RPT, RPT)],
                        out_hbm.at[cid, pl.ds(sid * RPT, RPT)])

    return k(dst_idx, zeros16)


def _sc_scatter(src_idx, dst_idx, table, zeros128):
    @functools.partial(
        pl.kernel,
        out_type=jax.ShapeDtypeStruct((NC, NNP, DD), jnp.float32),
        mesh=_mesh(),
        scratch_types=[
            pltpu.VMEM((GRP, CHUNK), jnp.int32),
            pltpu.VMEM((GRP, CHUNK), jnp.int32),
            pltpu.VMEM((2, CHUNK, DD), jnp.float32),
            pltpu.VMEM_SHARED((NNP, DD), jnp.float32),
            pltpu.SemaphoreType.DMA((2,)),
        ],
    )
    def k(src_hbm, dst_hbm, tab_hbm, zeros_hbm, out_hbm,
          srcv, dstv, rows, acc_sh, gsem):
        cid = lax.axis_index("c")
        sid = lax.axis_index("s")
        wid = sid * NC + cid
        pltpu.sync_copy(zeros_hbm.at[pl.ds(sid * RPT, RPT)],
                        acc_sh.at[pl.ds(sid * RPT, RPT)])
        plsc.subcore_barrier()

        def group(g, carry):
            pltpu.sync_copy(src_hbm.at[wid, g], srcv)
            pltpu.sync_copy(dst_hbm.at[wid, g], dstv)
            pltpu.async_copy(tab_hbm.at[srcv.at[0]], rows.at[0], gsem.at[0])

            def body(c, carry2):
                par = lax.rem(c, 2)
                nxt = lax.rem(c + 1, 2)

                @pl.when(c < GRP - 1)
                def _():
                    pltpu.async_copy(tab_hbm.at[srcv.at[c + 1]], rows.at[nxt],
                                     gsem.at[nxt])

                pltpu.make_async_copy(tab_hbm.at[srcv.at[c]], rows.at[par],
                                      gsem.at[par]).wait()
                pltpu.sync_copy(rows.at[par], acc_sh.at[dstv.at[c]], add=True)
                return carry2

            lax.fori_loop(0, GRP, body, 0)
            return carry

        lax.fori_loop(0, NGRP, group, 0)
        plsc.subcore_barrier()
        pltpu.sync_copy(acc_sh.at[pl.ds(sid * RPT, RPT)],
                        out_hbm.at[cid, pl.ds(sid * RPT, RPT)])

    return k(src_idx, dst_idx, table, zeros128)


def _moe_block(x, wg, W, b):
    r = x.shape[0]
    logits = jnp.dot(x, wg, preferred_element_type=jnp.float32)
    iota = lax.broadcasted_iota(jnp.int32, (r, NEXP), 1)
    v1 = jnp.max(logits, axis=1, keepdims=True)
    i1 = jnp.min(jnp.where(logits == v1, iota, NEXP), axis=1, keepdims=True)
    l2 = jnp.where(iota == i1, -jnp.inf, logits)
    v2 = jnp.max(l2, axis=1, keepdims=True)
    i2 = jnp.min(jnp.where(l2 == v2, iota, NEXP), axis=1, keepdims=True)
    a = 1.0 / (1.0 + jnp.exp(v2 - v1))
    gates = (jnp.where(iota == i1, a, 0.0)
             + jnp.where(iota == i2, 1.0 - a, 0.0))
    acc = jnp.zeros((r, DD), jnp.float32)
    for e in range(NEXP):
        y = jnp.dot(x, W[e], preferred_element_type=jnp.float32) + b[e][None, :]
        acc = acc + gates[:, e:e + 1] * y
    return acc


def _tc_moe1(x, wg, W, b):
    def body(x_ref, wg_ref, W_ref, b_ref, m_ref):
        m_ref[...] = _moe_block(x_ref[...], wg_ref[...], W_ref[...],
                                b_ref[...])

    grid = (NN // RB,)
    return pl.pallas_call(
        body,
        grid=grid,
        in_specs=[
            pl.BlockSpec((RB, DD), lambda i: (i, 0)),
            pl.BlockSpec((DD, NEXP), lambda i: (0, 0)),
            pl.BlockSpec((NEXP, DD, DD), lambda i: (0, 0, 0)),
            pl.BlockSpec((NEXP, DD), lambda i: (0, 0)),
        ],
        out_specs=pl.BlockSpec((RB, DD), lambda i: (i, 0)),
        out_shape=jax.ShapeDtypeStruct((NN, DD), jnp.float32),
    )(x, wg, W, b)


def _tc_scale(m, d0, d1):
    def body(m_ref, d0_ref, d1_ref, mp_ref, dinv_ref):
        dinv = lax.rsqrt(d0_ref[...] + d1_ref[...] + 1.0)
        mp_ref[...] = m_ref[...] * dinv
        dinv_ref[...] = dinv

    grid = (NN // RB,)
    return pl.pallas_call(
        body,
        grid=grid,
        in_specs=[
            pl.BlockSpec((RB, DD), lambda i: (i, 0)),
            pl.BlockSpec((RB, 1), lambda i: (i, 0)),
            pl.BlockSpec((RB, 1), lambda i: (i, 0)),
        ],
        out_specs=[
            pl.BlockSpec((RB, DD), lambda i: (i, 0)),
            pl.BlockSpec((RB, 1), lambda i: (i, 0)),
        ],
        out_shape=[
            jax.ShapeDtypeStruct((NN, DD), jnp.float32),
            jax.ShapeDtypeStruct((NN, 1), jnp.float32),
        ],
    )(m, d0, d1)


def _tc_moe2(p0, p1, mp1, dinv, wg, W, b):
    def body(p0_ref, p1_ref, mp1_ref, dinv_ref, wg_ref, W_ref, b_ref,
             out1_ref, mp2_ref):
        dinv = dinv_ref[...]
        out1 = (p0_ref[...] + p1_ref[...] + mp1_ref[...]) * dinv
        out1_ref[...] = out1
        m2 = _moe_block(out1, wg_ref[...], W_ref[...], b_ref[...])
        mp2_ref[...] = m2 * dinv

    grid = (NN // RB,)
    return pl.pallas_call(
        body,
        grid=grid,
        in_specs=[
            pl.BlockSpec((RB, DD), lambda i: (i, 0)),
            pl.BlockSpec((RB, DD), lambda i: (i, 0)),
            pl.BlockSpec((RB, DD), lambda i: (i, 0)),
            pl.BlockSpec((RB, 1), lambda i: (i, 0)),
            pl.BlockSpec((DD, NEXP), lambda i: (0, 0)),
            pl.BlockSpec((NEXP, DD, DD), lambda i: (0, 0, 0)),
            pl.BlockSpec((NEXP, DD), lambda i: (0, 0)),
        ],
        out_specs=[
            pl.BlockSpec((RB, DD), lambda i: (i, 0)),
            pl.BlockSpec((RB, DD), lambda i: (i, 0)),
        ],
        out_shape=[
            jax.ShapeDtypeStruct((NN, DD), jnp.float32),
            jax.ShapeDtypeStruct((NN, DD), jnp.float32),
        ],
    )(p0, p1, mp1, dinv, wg, W, b)


def _tc_comb(p0, p1, mp2, dinv):
    def body(p0_ref, p1_ref, mp2_ref, dinv_ref, out_ref):
        out_ref[...] = (p0_ref[...] + p1_ref[...] + mp2_ref[...]) * dinv_ref[...]

    grid = (NN // RB,)
    return pl.pallas_call(
        body,
        grid=grid,
        in_specs=[
            pl.BlockSpec((RB, DD), lambda i: (i, 0)),
            pl.BlockSpec((RB, DD), lambda i: (i, 0)),
            pl.BlockSpec((RB, DD), lambda i: (i, 0)),
            pl.BlockSpec((RB, 1), lambda i: (i, 0)),
        ],
        out_specs=pl.BlockSpec((RB, DD), lambda i: (i, 0)),
        out_shape=jax.ShapeDtypeStruct((NN, DD), jnp.float32),
    )(p0, p1, mp2, dinv)


def kernel(x, pos_edge_index, w_gate1, W1, b1, w_gate2, W2, b2):
    src = pos_edge_index[0].reshape(NW, NGRP, GRP, CHUNK)
    dst = pos_edge_index[1].reshape(NW, NGRP, GRP, CHUNK)
    zeros128 = jnp.zeros((NNP, DD), jnp.float32)
    zeros16 = jnp.zeros((NNP, DEG_W), jnp.float32)

    degp = _sc_degree(dst, zeros16)
    d0 = degp[0, :NN, :1]
    d1 = degp[1, :NN, :1]

    m1 = _tc_moe1(x, w_gate1, W1, b1)
    mp1, dinv = _tc_scale(m1, d0, d1)
    p1 = _sc_scatter(src, dst, mp1, zeros128)
    out1, mp2 = _tc_moe2(p1[0, :NN], p1[1, :NN], mp1, dinv, w_gate2, W2, b2)
    p2 = _sc_scatter(src, dst, mp2, zeros128)
    out2 = _tc_comb(p2[0, :NN], p2[1, :NN], mp2, dinv)
    return (out2, out1, out2)


# speedup vs baseline: 1.5179x; 1.0204x over previous
import functools

import jax
import jax.numpy as jnp
from jax import lax
from jax.experimental import pallas as pl
from jax.experimental.pallas import tpu as pltpu
from jax.experimental.pallas import tpu_sc as plsc

NN = 10000
EE = 320000
DD = 128
NEXP = 8
NC = 2
NS = 16
NW = NC * NS
EPW = EE // NW
CHUNK = 80
NCHUNK = EPW // CHUNK
GRP = 25
NGRP = NCHUNK // GRP
NNP = 10240
RPT = NNP // NS
DEG_W = 16
RB = 2000


def _mesh():
    return plsc.VectorSubcoreMesh(core_axis_name="c", subcore_axis_name="s")


def _sc_degree(dst_idx, zeros16):
    @functools.partial(
        pl.kernel,
        out_type=jax.ShapeDtypeStruct((NC, NNP, DEG_W), jnp.float32),
        mesh=_mesh(),
        scratch_types=[
            pltpu.VMEM((GRP, CHUNK), jnp.int32),
            pltpu.VMEM((CHUNK, DEG_W), jnp.float32),
            pltpu.VMEM_SHARED((NNP, DEG_W), jnp.float32),
            pltpu.SemaphoreType.DMA,
        ],
    )
    def k(dst_hbm, zeros_hbm, out_hbm, idx_v, ones_v, acc_sh, dsem):
        cid = lax.axis_index("c")
        sid = lax.axis_index("s")
        wid = sid * NC + cid
        one16 = jnp.full((DEG_W,), 1.0, dtype=jnp.float32)
        for i in range(CHUNK):
            ones_v[i, :] = one16
        pltpu.sync_copy(zeros_hbm.at[pl.ds(sid * RPT, RPT)],
                        acc_sh.at[pl.ds(sid * RPT, RPT)])
        plsc.subcore_barrier()

        def group(g, carry):
            pltpu.sync_copy(dst_hbm.at[wid, g], idx_v)

            def fire(c, carry2):
                pltpu.async_copy(ones_v, acc_sh.at[idx_v.at[c]], dsem,
                                 add=True)
                return carry2

            lax.fori_loop(0, GRP, fire, 0)

            def drain(c, carry2):
                pltpu.make_async_copy(ones_v, acc_sh.at[idx_v.at[c]],
                                      dsem).wait()
                return carry2

            lax.fori_loop(0, GRP, drain, 0)
            return carry

        lax.fori_loop(0, NGRP, group, 0)
        plsc.subcore_barrier()
        pltpu.sync_copy(acc_sh.at[pl.ds(sid * RPT, RPT)],
                        out_hbm.at[cid, pl.ds(sid * RPT, RPT)])

    return k(dst_idx, zeros16)


def _sc_scatter(src_idx, dst_idx, table, zeros128):
    @functools.partial(
        pl.kernel,
        out_type=jax.ShapeDtypeStruct((NC, NNP, DD), jnp.float32),
        mesh=_mesh(),
        scratch_types=[
            pltpu.VMEM((GRP, CHUNK), jnp.int32),
            pltpu.VMEM((GRP, CHUNK), jnp.int32),
            pltpu.VMEM((2, CHUNK, DD), jnp.float32),
            pltpu.VMEM_SHARED((NNP, DD), jnp.float32),
            pltpu.SemaphoreType.DMA((2,)),
        ],
    )
    def k(src_hbm, dst_hbm, tab_hbm, zeros_hbm, out_hbm,
          srcv, dstv, rows, acc_sh, gsem):
        cid = lax.axis_index("c")
        sid = lax.axis_index("s")
        wid = sid * NC + cid
        pltpu.sync_copy(zeros_hbm.at[pl.ds(sid * RPT, RPT)],
                        acc_sh.at[pl.ds(sid * RPT, RPT)])
        plsc.subcore_barrier()

        def group(g, carry):
            pltpu.sync_copy(src_hbm.at[wid, g], srcv)
            pltpu.sync_copy(dst_hbm.at[wid, g], dstv)
            pltpu.async_copy(tab_hbm.at[srcv.at[0]], rows.at[0], gsem.at[0])

            def body(c, carry2):
                par = lax.rem(c, 2)
                nxt = lax.rem(c + 1, 2)

                @pl.when(c < GRP - 1)
                def _():
                    pltpu.async_copy(tab_hbm.at[srcv.at[c + 1]], rows.at[nxt],
                                     gsem.at[nxt])

                pltpu.make_async_copy(tab_hbm.at[srcv.at[c]], rows.at[par],
                                      gsem.at[par]).wait()
                pltpu.sync_copy(rows.at[par], acc_sh.at[dstv.at[c]], add=True)
                return carry2

            lax.fori_loop(0, GRP, body, 0)
            return carry

        lax.fori_loop(0, NGRP, group, 0)
        plsc.subcore_barrier()
        pltpu.sync_copy(acc_sh.at[pl.ds(sid * RPT, RPT)],
                        out_hbm.at[cid, pl.ds(sid * RPT, RPT)])

    return k(src_idx, dst_idx, table, zeros128)


def _moe_block(x, wg, W, b):
    r = x.shape[0]
    logits = jnp.dot(x, wg, preferred_element_type=jnp.float32)
    iota = lax.broadcasted_iota(jnp.int32, (r, NEXP), 1)
    v1 = jnp.max(logits, axis=1, keepdims=True)
    i1 = jnp.min(jnp.where(logits == v1, iota, NEXP), axis=1, keepdims=True)
    l2 = jnp.where(iota == i1, -jnp.inf, logits)
    v2 = jnp.max(l2, axis=1, keepdims=True)
    i2 = jnp.min(jnp.where(l2 == v2, iota, NEXP), axis=1, keepdims=True)
    a = 1.0 / (1.0 + jnp.exp(v2 - v1))
    gates = (jnp.where(iota == i1, a, 0.0)
             + jnp.where(iota == i2, 1.0 - a, 0.0))
    acc = jnp.zeros((r, DD), jnp.float32)
    for e in range(NEXP):
        y = jnp.dot(x, W[e], preferred_element_type=jnp.float32) + b[e][None, :]
        acc = acc + gates[:, e:e + 1] * y
    return acc


def _tc_moe1(x, wg, W, b):
    def body(x_ref, wg_ref, W_ref, b_ref, m_ref):
        m_ref[...] = _moe_block(x_ref[...], wg_ref[...], W_ref[...],
                                b_ref[...])

    grid = (NN // RB,)
    return pl.pallas_call(
        body,
        grid=grid,
        in_specs=[
            pl.BlockSpec((RB, DD), lambda i: (i, 0)),
            pl.BlockSpec((DD, NEXP), lambda i: (0, 0)),
            pl.BlockSpec((NEXP, DD, DD), lambda i: (0, 0, 0)),
            pl.BlockSpec((NEXP, DD), lambda i: (0, 0)),
        ],
        out_specs=pl.BlockSpec((RB, DD), lambda i: (i, 0)),
        out_shape=jax.ShapeDtypeStruct((NN, DD), jnp.float32),
    )(x, wg, W, b)


def _tc_scale(m, d0, d1):
    def body(m_ref, d0_ref, d1_ref, mp_ref, dinv_ref):
        dinv = lax.rsqrt(d0_ref[...] + d1_ref[...] + 1.0)
        mp_ref[...] = m_ref[...] * dinv
        dinv_ref[...] = dinv

    grid = (NN // RB,)
    return pl.pallas_call(
        body,
        grid=grid,
        in_specs=[
            pl.BlockSpec((RB, DD), lambda i: (i, 0)),
            pl.BlockSpec((RB, 1), lambda i: (i, 0)),
            pl.BlockSpec((RB, 1), lambda i: (i, 0)),
        ],
        out_specs=[
            pl.BlockSpec((RB, DD), lambda i: (i, 0)),
            pl.BlockSpec((RB, 1), lambda i: (i, 0)),
        ],
        out_shape=[
            jax.ShapeDtypeStruct((NN, DD), jnp.float32),
            jax.ShapeDtypeStruct((NN, 1), jnp.float32),
        ],
    )(m, d0, d1)


def _tc_moe2(p0, p1, mp1, dinv, wg, W, b):
    def body(p0_ref, p1_ref, mp1_ref, dinv_ref, wg_ref, W_ref, b_ref,
             out1_ref, mp2_ref):
        dinv = dinv_ref[...]
        out1 = (p0_ref[...] + p1_ref[...] + mp1_ref[...]) * dinv
        out1_ref[...] = out1
        m2 = _moe_block(out1, wg_ref[...], W_ref[...], b_ref[...])
        mp2_ref[...] = m2 * dinv

    grid = (NN // RB,)
    return pl.pallas_call(
        body,
        grid=grid,
        in_specs=[
            pl.BlockSpec((RB, DD), lambda i: (i, 0)),
            pl.BlockSpec((RB, DD), lambda i: (i, 0)),
            pl.BlockSpec((RB, DD), lambda i: (i, 0)),
            pl.BlockSpec((RB, 1), lambda i: (i, 0)),
            pl.BlockSpec((DD, NEXP), lambda i: (0, 0)),
            pl.BlockSpec((NEXP, DD, DD), lambda i: (0, 0, 0)),
            pl.BlockSpec((NEXP, DD), lambda i: (0, 0)),
        ],
        out_specs=[
            pl.BlockSpec((RB, DD), lambda i: (i, 0)),
            pl.BlockSpec((RB, DD), lambda i: (i, 0)),
        ],
        out_shape=[
            jax.ShapeDtypeStruct((NN, DD), jnp.float32),
            jax.ShapeDtypeStruct((NN, DD), jnp.float32),
        ],
    )(p0, p1, mp1, dinv, wg, W, b)


def _tc_comb(p0, p1, mp2, dinv):
    def body(p0_ref, p1_ref, mp2_ref, dinv_ref, out_ref):
        out_ref[...] = (p0_ref[...] + p1_ref[...] + mp2_ref[...]) * dinv_ref[...]

    grid = (NN // RB,)
    return pl.pallas_call(
        body,
        grid=grid,
        in_specs=[
            pl.BlockSpec((RB, DD), lambda i: (i, 0)),
            pl.BlockSpec((RB, DD), lambda i: (i, 0)),
            pl.BlockSpec((RB, DD), lambda i: (i, 0)),
            pl.BlockSpec((RB, 1), lambda i: (i, 0)),
        ],
        out_specs=pl.BlockSpec((RB, DD), lambda i: (i, 0)),
        out_shape=jax.ShapeDtypeStruct((NN, DD), jnp.float32),
    )(p0, p1, mp2, dinv)


def kernel(x, pos_edge_index, w_gate1, W1, b1, w_gate2, W2, b2):
    src = pos_edge_index[0].reshape(NW, NGRP, GRP, CHUNK)
    dst = pos_edge_index[1].reshape(NW, NGRP, GRP, CHUNK)
    zeros128 = jnp.zeros((NNP, DD), jnp.float32)
    zeros16 = jnp.zeros((NNP, DEG_W), jnp.float32)

    degp = _sc_degree(dst, zeros16)
    d0 = degp[0, :NN, :1]
    d1 = degp[1, :NN, :1]

    m1 = _tc_moe1(x, w_gate1, W1, b1)
    mp1, dinv = _tc_scale(m1, d0, d1)
    p1 = _sc_scatter(src, dst, mp1, zeros128)
    out1, mp2 = _tc_moe2(p1[0, :NN], p1[1, :NN], mp1, dinv, w_gate2, W2, b2)
    p2 = _sc_scatter(src, dst, mp2, zeros128)
    out2 = _tc_comb(p2[0, :NN], p2[1, :NN], mp2, dinv)
    return (out2, out1, out2)


# speedup vs baseline: 1.5665x; 1.0320x over previous
import functools

import jax
import jax.numpy as jnp
from jax import lax
from jax.experimental import pallas as pl
from jax.experimental.pallas import tpu as pltpu
from jax.experimental.pallas import tpu_sc as plsc

NN = 10000
EE = 320000
DD = 128
NEXP = 8
NC = 2
NS = 16
NW = NC * NS
EPW = EE // NW
CHUNK = 80
NCHUNK = EPW // CHUNK
GRP = 25
NGRP = NCHUNK // GRP
NNP = 10240
RPT = NNP // NS
DEG_W = 16
RB = 2000


def _mesh():
    return plsc.VectorSubcoreMesh(core_axis_name="c", subcore_axis_name="s")


def _sc_degree(dst_idx, zeros16):
    @functools.partial(
        pl.kernel,
        out_type=jax.ShapeDtypeStruct((NC, NNP, DEG_W), jnp.float32),
        mesh=_mesh(),
        scratch_types=[
            pltpu.VMEM((GRP, CHUNK), jnp.int32),
            pltpu.VMEM((CHUNK, DEG_W), jnp.float32),
            pltpu.VMEM_SHARED((NNP, DEG_W), jnp.float32),
            pltpu.SemaphoreType.DMA,
        ],
    )
    def k(dst_hbm, zeros_hbm, out_hbm, idx_v, ones_v, acc_sh, dsem):
        cid = lax.axis_index("c")
        sid = lax.axis_index("s")
        wid = sid * NC + cid
        one16 = jnp.full((DEG_W,), 1.0, dtype=jnp.float32)
        for i in range(CHUNK):
            ones_v[i, :] = one16
        pltpu.sync_copy(zeros_hbm.at[pl.ds(sid * RPT, RPT)],
                        acc_sh.at[pl.ds(sid * RPT, RPT)])
        plsc.subcore_barrier()

        def group(g, carry):
            pltpu.sync_copy(dst_hbm.at[wid, g], idx_v)

            def fire(c, carry2):
                pltpu.async_copy(ones_v, acc_sh.at[idx_v.at[c]], dsem,
                                 add=True)
                return carry2

            lax.fori_loop(0, GRP, fire, 0)

            def drain(c, carry2):
                pltpu.make_async_copy(ones_v, acc_sh.at[idx_v.at[c]],
                                      dsem).wait()
                return carry2

            lax.fori_loop(0, GRP, drain, 0)
            return carry

        lax.fori_loop(0, NGRP, group, 0)
        plsc.subcore_barrier()
        pltpu.sync_copy(acc_sh.at[pl.ds(sid * RPT, RPT)],
                        out_hbm.at[cid, pl.ds(sid * RPT, RPT)])

    return k(dst_idx, zeros16)


def _sc_scatter(src_idx, dst_idx, table, zeros128):
    @functools.partial(
        pl.kernel,
        out_type=jax.ShapeDtypeStruct((NC, NNP, DD), jnp.float32),
        mesh=_mesh(),
        scratch_types=[
            pltpu.VMEM((2, GRP, CHUNK), jnp.int32),
            pltpu.VMEM((2, GRP, CHUNK), jnp.int32),
            pltpu.VMEM((2, CHUNK, DD), jnp.float32),
            pltpu.VMEM_SHARED((NNP, DD), jnp.float32),
            pltpu.SemaphoreType.DMA((2,)),
            pltpu.SemaphoreType.DMA((2,)),
        ],
    )
    def k(src_hbm, dst_hbm, tab_hbm, zeros_hbm, out_hbm,
          srcv, dstv, rows, acc_sh, gsem, stsem):
        cid = lax.axis_index("c")
        sid = lax.axis_index("s")
        wid = sid * NC + cid
        pltpu.async_copy(src_hbm.at[wid, 0], srcv.at[0], stsem.at[0])
        pltpu.async_copy(dst_hbm.at[wid, 0], dstv.at[0], stsem.at[0])
        pltpu.async_copy(src_hbm.at[wid, 1], srcv.at[1], stsem.at[1])
        pltpu.async_copy(dst_hbm.at[wid, 1], dstv.at[1], stsem.at[1])
        pltpu.sync_copy(zeros_hbm.at[pl.ds(sid * RPT, RPT)],
                        acc_sh.at[pl.ds(sid * RPT, RPT)])
        plsc.subcore_barrier()

        for g in range(NGRP):
            slot = g % 2
            sv = srcv.at[slot]
            dv = dstv.at[slot]
            pltpu.make_async_copy(src_hbm.at[wid, g], sv, stsem.at[slot]).wait()
            pltpu.make_async_copy(dst_hbm.at[wid, g], dv, stsem.at[slot]).wait()
            pltpu.async_copy(tab_hbm.at[sv.at[0]], rows.at[0], gsem.at[0])

            def body(c, carry2, sv=sv, dv=dv):
                par = lax.rem(c, 2)
                nxt = lax.rem(c + 1, 2)

                @pl.when(c < GRP - 1)
                def _():
                    pltpu.async_copy(tab_hbm.at[sv.at[c + 1]], rows.at[nxt],
                                     gsem.at[nxt])

                pltpu.make_async_copy(tab_hbm.at[sv.at[c]], rows.at[par],
                                      gsem.at[par]).wait()
                pltpu.sync_copy(rows.at[par], acc_sh.at[dv.at[c]], add=True)
                return carry2

            lax.fori_loop(0, GRP, body, 0)
            if g + 2 < NGRP:
                pltpu.async_copy(src_hbm.at[wid, g + 2], sv, stsem.at[slot])
                pltpu.async_copy(dst_hbm.at[wid, g + 2], dv, stsem.at[slot])
        plsc.subcore_barrier()
        pltpu.sync_copy(acc_sh.at[pl.ds(sid * RPT, RPT)],
                        out_hbm.at[cid, pl.ds(sid * RPT, RPT)])

    return k(src_idx, dst_idx, table, zeros128)


def _moe_block(x, wg, W, b):
    r = x.shape[0]
    logits = jnp.dot(x, wg, preferred_element_type=jnp.float32)
    iota = lax.broadcasted_iota(jnp.int32, (r, NEXP), 1)
    v1 = jnp.max(logits, axis=1, keepdims=True)
    i1 = jnp.min(jnp.where(logits == v1, iota, NEXP), axis=1, keepdims=True)
    l2 = jnp.where(iota == i1, -jnp.inf, logits)
    v2 = jnp.max(l2, axis=1, keepdims=True)
    i2 = jnp.min(jnp.where(l2 == v2, iota, NEXP), axis=1, keepdims=True)
    a = 1.0 / (1.0 + jnp.exp(v2 - v1))
    gates = (jnp.where(iota == i1, a, 0.0)
             + jnp.where(iota == i2, 1.0 - a, 0.0))
    acc = jnp.zeros((r, DD), jnp.float32)
    for e in range(NEXP):
        y = jnp.dot(x, W[e], preferred_element_type=jnp.float32) + b[e][None, :]
        acc = acc + gates[:, e:e + 1] * y
    return acc


def _tc_moe1(x, wg, W, b):
    def body(x_ref, wg_ref, W_ref, b_ref, m_ref):
        m_ref[...] = _moe_block(x_ref[...], wg_ref[...], W_ref[...],
                                b_ref[...])

    grid = (NN // RB,)
    return pl.pallas_call(
        body,
        grid=grid,
        in_specs=[
            pl.BlockSpec((RB, DD), lambda i: (i, 0)),
            pl.BlockSpec((DD, NEXP), lambda i: (0, 0)),
            pl.BlockSpec((NEXP, DD, DD), lambda i: (0, 0, 0)),
            pl.BlockSpec((NEXP, DD), lambda i: (0, 0)),
        ],
        out_specs=pl.BlockSpec((RB, DD), lambda i: (i, 0)),
        out_shape=jax.ShapeDtypeStruct((NN, DD), jnp.float32),
    )(x, wg, W, b)


def _tc_scale(m, d0, d1):
    def body(m_ref, d0_ref, d1_ref, mp_ref, dinv_ref):
        dinv = lax.rsqrt(d0_ref[...] + d1_ref[...] + 1.0)
        mp_ref[...] = m_ref[...] * dinv
        dinv_ref[...] = dinv

    grid = (NN // RB,)
    return pl.pallas_call(
        body,
        grid=grid,
        in_specs=[
            pl.BlockSpec((RB, DD), lambda i: (i, 0)),
            pl.BlockSpec((RB, 1), lambda i: (i, 0)),
            pl.BlockSpec((RB, 1), lambda i: (i, 0)),
        ],
        out_specs=[
            pl.BlockSpec((RB, DD), lambda i: (i, 0)),
            pl.BlockSpec((RB, 1), lambda i: (i, 0)),
        ],
        out_shape=[
            jax.ShapeDtypeStruct((NN, DD), jnp.float32),
            jax.ShapeDtypeStruct((NN, 1), jnp.float32),
        ],
    )(m, d0, d1)


def _tc_moe2(p0, p1, mp1, dinv, wg, W, b):
    def body(p0_ref, p1_ref, mp1_ref, dinv_ref, wg_ref, W_ref, b_ref,
             out1_ref, mp2_ref):
        dinv = dinv_ref[...]
        out1 = (p0_ref[...] + p1_ref[...] + mp1_ref[...]) * dinv
        out1_ref[...] = out1
        m2 = _moe_block(out1, wg_ref[...], W_ref[...], b_ref[...])
        mp2_ref[...] = m2 * dinv

    grid = (NN // RB,)
    return pl.pallas_call(
        body,
        grid=grid,
        in_specs=[
            pl.BlockSpec((RB, DD), lambda i: (i, 0)),
            pl.BlockSpec((RB, DD), lambda i: (i, 0)),
            pl.BlockSpec((RB, DD), lambda i: (i, 0)),
            pl.BlockSpec((RB, 1), lambda i: (i, 0)),
            pl.BlockSpec((DD, NEXP), lambda i: (0, 0)),
            pl.BlockSpec((NEXP, DD, DD), lambda i: (0, 0, 0)),
            pl.BlockSpec((NEXP, DD), lambda i: (0, 0)),
        ],
        out_specs=[
            pl.BlockSpec((RB, DD), lambda i: (i, 0)),
            pl.BlockSpec((RB, DD), lambda i: (i, 0)),
        ],
        out_shape=[
            jax.ShapeDtypeStruct((NN, DD), jnp.float32),
            jax.ShapeDtypeStruct((NN, DD), jnp.float32),
        ],
    )(p0, p1, mp1, dinv, wg, W, b)


def _tc_comb(p0, p1, mp2, dinv):
    def body(p0_ref, p1_ref, mp2_ref, dinv_ref, out_ref):
        out_ref[...] = (p0_ref[...] + p1_ref[...] + mp2_ref[...]) * dinv_ref[...]

    grid = (NN // RB,)
    return pl.pallas_call(
        body,
        grid=grid,
        in_specs=[
            pl.BlockSpec((RB, DD), lambda i: (i, 0)),
            pl.BlockSpec((RB, DD), lambda i: (i, 0)),
            pl.BlockSpec((RB, DD), lambda i: (i, 0)),
            pl.BlockSpec((RB, 1), lambda i: (i, 0)),
        ],
        out_specs=pl.BlockSpec((RB, DD), lambda i: (i, 0)),
        out_shape=jax.ShapeDtypeStruct((NN, DD), jnp.float32),
    )(p0, p1, mp2, dinv)


def kernel(x, pos_edge_index, w_gate1, W1, b1, w_gate2, W2, b2):
    src = pos_edge_index[0].reshape(NW, NGRP, GRP, CHUNK)
    dst = pos_edge_index[1].reshape(NW, NGRP, GRP, CHUNK)
    zeros128 = jnp.zeros((NNP, DD), jnp.float32)
    zeros16 = jnp.zeros((NNP, DEG_W), jnp.float32)

    degp = _sc_degree(dst, zeros16)
    d0 = degp[0, :NN, :1]
    d1 = degp[1, :NN, :1]

    m1 = _tc_moe1(x, w_gate1, W1, b1)
    mp1, dinv = _tc_scale(m1, d0, d1)
    p1 = _sc_scatter(src, dst, mp1, zeros128)
    out1, mp2 = _tc_moe2(p1[0, :NN], p1[1, :NN], mp1, dinv, w_gate2, W2, b2)
    p2 = _sc_scatter(src, dst, mp2, zeros128)
    out2 = _tc_comb(p2[0, :NN], p2[1, :NN], mp2, dinv)
    return (out2, out1, out2)
